# Initial kernel scaffold; baseline (speedup 1.0000x reference)
import functools

import jax
import jax.numpy as jnp
from jax import lax
from jax.experimental import pallas as pl
from jax.experimental.pallas import tpu as pltpu
from jax.experimental.pallas import tpu_sc as plsc

SEQ = 200
D = 64
BATCH = 4096
NC = 2
NS = 16
NW = NC * NS
TOTAL = BATCH * SEQ
PER_W = TOTAL // NW
CHUNK = 128
NCHUNK = PER_W // CHUNK


def _sc_embed(xflat, table, pe2):
    mesh = plsc.VectorSubcoreMesh(
        core_axis_name="c", subcore_axis_name="s",
        num_cores=NC, num_subcores=NS)

    @functools.partial(
        pl.kernel,
        out_type=jax.ShapeDtypeStruct((TOTAL, D), jnp.float32),
        mesh=mesh,
        scratch_types=[
            pltpu.VMEM((SEQ, D), jnp.float32),
            pltpu.VMEM((CHUNK,), jnp.int32),
            pltpu.VMEM((CHUNK, D), jnp.float32),
            pltpu.SemaphoreType.DMA,
        ],
    )
    def k(x_hbm, tab_hbm, pe_hbm, out_hbm, pe_v, idx_v, rows_v, sem):
        wid = lax.axis_index("s") * NC + lax.axis_index("c")
        base = wid * PER_W
        pltpu.sync_copy(pe_hbm, pe_v)

        def chunk_body(g, carry):
            r0 = base + g * CHUNK
            pltpu.sync_copy(x_hbm.at[pl.ds(r0, CHUNK)], idx_v)
            pltpu.async_copy(tab_hbm.at[idx_v], rows_v, sem).wait()
            p0 = lax.rem(g * CHUNK, SEQ)

            def row_body(i, c2):
                pr = lax.rem(p0 + i, SEQ)
                for l in range(D // 16):
                    sl = pl.ds(l * 16, 16)
                    rows_v[i, sl] += pe_v[pr, sl]
                return c2

            lax.fori_loop(0, CHUNK, row_body, 0)
            pltpu.sync_copy(rows_v, out_hbm.at[pl.ds(r0, CHUNK)])
            return carry

        lax.fori_loop(0, NCHUNK, chunk_body, 0)

    return k(xflat, table, pe2)


def kernel(x, table, pe):
    xflat = x.reshape(TOTAL).astype(jnp.int32)
    pe2 = pe.reshape(SEQ, D)
    out = _sc_embed(xflat, table, pe2)
    return out.reshape(BATCH, SEQ, D)


# speedup vs baseline: 2.1441x; 2.1441x over previous
import functools

import jax
import jax.numpy as jnp
from jax import lax
from jax.experimental import pallas as pl
from jax.experimental.pallas import tpu as pltpu
from jax.experimental.pallas import tpu_sc as plsc

SEQ = 200
D = 64
BATCH = 4096
NC = 2
NS = 16
NW = NC * NS
TOTAL = BATCH * SEQ
PER_W = TOTAL // NW
CHUNK = 128
NCHUNK = PER_W // CHUNK


def _sc_embed(xflat, table, pe2):
    mesh = plsc.VectorSubcoreMesh(
        core_axis_name="c", subcore_axis_name="s",
        num_cores=NC, num_subcores=NS)

    @functools.partial(
        pl.kernel,
        out_type=jax.ShapeDtypeStruct((TOTAL, D), jnp.float32),
        mesh=mesh,
        scratch_types=[
            pltpu.VMEM((SEQ, D), jnp.float32),
            pltpu.VMEM((CHUNK,), jnp.int32),
            pltpu.VMEM((CHUNK, D), jnp.float32),
            pltpu.SemaphoreType.DMA,
        ],
        compiler_params=pltpu.CompilerParams(use_tc_tiling_on_sc=False),
    )
    def k(x_hbm, tab_hbm, pe_hbm, out_hbm, pe_v, idx_v, rows_v, sem):
        wid = lax.axis_index("s") * NC + lax.axis_index("c")
        base = wid * PER_W
        pltpu.sync_copy(pe_hbm, pe_v)

        def chunk_body(g, carry):
            r0 = base + g * CHUNK
            pltpu.sync_copy(x_hbm.at[pl.ds(r0, CHUNK)], idx_v)
            pltpu.async_copy(tab_hbm.at[idx_v], rows_v, sem).wait()
            p0 = lax.rem(g * CHUNK, SEQ)

            def row_body(i, c2):
                pr = lax.rem(p0 + i, SEQ)
                for l in range(D // 16):
                    sl = pl.ds(l * 16, 16)
                    rows_v[i, sl] += pe_v[pr, sl]
                return c2

            lax.fori_loop(0, CHUNK, row_body, 0)
            pltpu.sync_copy(rows_v, out_hbm.at[pl.ds(r0, CHUNK)])
            return carry

        lax.fori_loop(0, NCHUNK, chunk_body, 0)

    return k(xflat, table, pe2)


def kernel(x, table, pe):
    xflat = x.reshape(TOTAL).astype(jnp.int32)
    pe2 = pe.reshape(SEQ, D)
    out = _sc_embed(xflat, table, pe2)
    return out.reshape(BATCH, SEQ, D)


# speedup vs baseline: 4.2236x; 1.9699x over previous
import functools

import jax
import jax.numpy as jnp
from jax import lax
from jax.experimental import pallas as pl
from jax.experimental.pallas import tpu as pltpu
from jax.experimental.pallas import tpu_sc as plsc

SEQ = 200
D = 64
BATCH = 4096
NC = 2
NS = 16
NW = NC * NS
TOTAL = BATCH * SEQ
PER_W = TOTAL // NW
CHUNK = 2 * SEQ
NCHUNK = PER_W // CHUNK
NGATH = 4
GIDX = CHUNK // NGATH
NBUF = 3


def _sc_embed(x2, table, pe2x):
    mesh = plsc.VectorSubcoreMesh(
        core_axis_name="c", subcore_axis_name="s",
        num_cores=NC, num_subcores=NS)

    @functools.partial(
        pl.kernel,
        out_type=jax.ShapeDtypeStruct((TOTAL, D), jnp.float32),
        mesh=mesh,
        scratch_types=[
            pltpu.VMEM_SHARED((CHUNK, D), jnp.float32),
            pltpu.VMEM((NBUF, NGATH, GIDX), jnp.int32),
            pltpu.VMEM((NBUF, CHUNK, D), jnp.float32),
            pltpu.SemaphoreType.DMA((NBUF,)),
            pltpu.SemaphoreType.DMA((NBUF,)),
            pltpu.SemaphoreType.DMA((NBUF,)),
            pltpu.SemaphoreType.DMA((NBUF,)),
        ],
        compiler_params=pltpu.CompilerParams(use_tc_tiling_on_sc=False),
    )
    def k(x_hbm, tab_hbm, pe_hbm, out_hbm,
          pe_v, idx_v, rows_v, s_idx, s_fill, s_gath, s_out):
        wid = lax.axis_index("s") * NC + lax.axis_index("c")
        base = wid * PER_W

        @pl.when(lax.axis_index("s") == 0)
        def _():
            pltpu.sync_copy(pe_hbm, pe_v)
        plsc.subcore_barrier()

        def start_chunk(g):
            b = lax.rem(g, NBUF)
            r0 = base + g * CHUNK
            pltpu.async_copy(
                x_hbm.at[pl.ds(r0 // GIDX, NGATH)], idx_v.at[b], s_idx.at[b])
            pltpu.async_copy(pe_v, rows_v.at[b], s_fill.at[b])

        def gather_chunk(g):
            b = lax.rem(g, NBUF)
            pltpu.make_async_copy(
                x_hbm.at[pl.ds(0, NGATH)], idx_v.at[b], s_idx.at[b]).wait()
            pltpu.make_async_copy(pe_v, rows_v.at[b], s_fill.at[b]).wait()
            for j in range(NGATH):
                pltpu.async_copy(
                    tab_hbm.at[idx_v.at[b, j]],
                    rows_v.at[b, pl.ds(j * GIDX, GIDX)],
                    s_gath.at[b], add=True)

        def store_chunk(g):
            b = lax.rem(g, NBUF)
            r0 = base + g * CHUNK
            pltpu.make_async_copy(
                tab_hbm.at[idx_v.at[b, 0]], rows_v.at[b], s_gath.at[b]).wait()
            pltpu.async_copy(
                rows_v.at[b], out_hbm.at[pl.ds(r0, CHUNK)], s_out.at[b])

        def wait_store(g):
            b = lax.rem(g, NBUF)
            r0 = base + g * CHUNK
            pltpu.make_async_copy(
                rows_v.at[b], out_hbm.at[pl.ds(r0, CHUNK)], s_out.at[b]).wait()

        def body(g, carry):
            @pl.when(jnp.logical_and(g >= 2, g - 2 < NCHUNK))
            def _():
                store_chunk(g - 2)

            @pl.when(jnp.logical_and(g >= 1, g - 1 < NCHUNK))
            def _():
                gather_chunk(g - 1)

            @pl.when(g < NCHUNK)
            def _():
                @pl.when(g >= NBUF)
                def _():
                    wait_store(g - NBUF)
                start_chunk(g)
            return carry

        lax.fori_loop(0, NCHUNK + 2, body, 0)
        for j in range(min(NBUF, NCHUNK)):
            wait_store(NCHUNK - 1 - j)

    return k(x2, table, pe2x)


def kernel(x, table, pe):
    x2 = x.reshape(TOTAL // GIDX, GIDX).astype(jnp.int32)
    pe2 = pe.reshape(SEQ, D)
    pe2x = jnp.concatenate([pe2, pe2], axis=0)
    out = _sc_embed(x2, table, pe2x)
    return out.reshape(BATCH, SEQ, D)


# speedup vs baseline: 4.2410x; 1.0041x over previous
import functools

import jax
import jax.numpy as jnp
from jax import lax
from jax.experimental import pallas as pl
from jax.experimental.pallas import tpu as pltpu
from jax.experimental.pallas import tpu_sc as plsc

SEQ = 200
D = 64
BATCH = 4096
NC = 2
NS = 16
NW = NC * NS
ROWS_PER_W = BATCH // NW
RPC = 2
NCHUNK = ROWS_PER_W // RPC
SPLITS = ((0, 128), (128, 72))
NBUF = 3


def _sc_embed(x, table, pe2):
    mesh = plsc.VectorSubcoreMesh(
        core_axis_name="c", subcore_axis_name="s",
        num_cores=NC, num_subcores=NS)

    @functools.partial(
        pl.kernel,
        out_type=jax.ShapeDtypeStruct((BATCH, SEQ, D), jnp.float32),
        mesh=mesh,
        scratch_types=[
            pltpu.VMEM_SHARED((SEQ, D), jnp.float32),
            pltpu.VMEM((NBUF, RPC, SEQ), jnp.int32),
            pltpu.VMEM((NBUF, RPC, SEQ, D), jnp.float32),
            pltpu.SemaphoreType.DMA((NBUF,)),
            pltpu.SemaphoreType.DMA((NBUF,)),
            pltpu.SemaphoreType.DMA((NBUF,)),
            pltpu.SemaphoreType.DMA((NBUF,)),
        ],
        compiler_params=pltpu.CompilerParams(use_tc_tiling_on_sc=False),
    )
    def k(x_hbm, tab_hbm, pe_hbm, out_hbm,
          pe_v, idx_v, rows_v, s_idx, s_fill, s_gath, s_out):
        wid = lax.axis_index("s") * NC + lax.axis_index("c")
        base = wid * ROWS_PER_W

        @pl.when(lax.axis_index("s") == 0)
        def _():
            pltpu.sync_copy(pe_hbm, pe_v)
        plsc.subcore_barrier()

        def start_chunk(g):
            b = lax.rem(g, NBUF)
            r0 = base + g * RPC
            pltpu.async_copy(x_hbm.at[pl.ds(r0, RPC)], idx_v.at[b],
                             s_idx.at[b])
            for i in range(RPC):
                pltpu.async_copy(pe_v, rows_v.at[b, i], s_fill.at[b])

        def gather_chunk(g):
            b = lax.rem(g, NBUF)
            pltpu.make_async_copy(
                x_hbm.at[pl.ds(0, RPC)], idx_v.at[b], s_idx.at[b]).wait()
            for i in range(RPC):
                pltpu.make_async_copy(
                    pe_v, rows_v.at[b, i], s_fill.at[b]).wait()
            for i in range(RPC):
                for (o, n) in SPLITS:
                    pltpu.async_copy(
                        tab_hbm.at[idx_v.at[b, i, pl.ds(o, n)]],
                        rows_v.at[b, i, pl.ds(o, n)],
                        s_gath.at[b], add=True)

        def store_chunk(g):
            b = lax.rem(g, NBUF)
            r0 = base + g * RPC
            for i in range(RPC):
                for (o, n) in SPLITS:
                    pltpu.make_async_copy(
                        tab_hbm.at[idx_v.at[b, i, pl.ds(o, n)]],
                        rows_v.at[b, i, pl.ds(o, n)],
                        s_gath.at[b]).wait()
            pltpu.async_copy(rows_v.at[b], out_hbm.at[pl.ds(r0, RPC)],
                             s_out.at[b])

        def wait_store(g):
            b = lax.rem(g, NBUF)
            r0 = base + g * RPC
            pltpu.make_async_copy(
                rows_v.at[b], out_hbm.at[pl.ds(r0, RPC)], s_out.at[b]).wait()

        def body(g, carry):
            @pl.when(jnp.logical_and(g >= 2, g - 2 < NCHUNK))
            def _():
                store_chunk(g - 2)

            @pl.when(jnp.logical_and(g >= 1, g - 1 < NCHUNK))
            def _():
                gather_chunk(g - 1)

            @pl.when(g < NCHUNK)
            def _():
                @pl.when(g >= NBUF)
                def _():
                    wait_store(g - NBUF)
                start_chunk(g)
            return carry

        lax.fori_loop(0, NCHUNK + 2, body, 0)
        for j in range(min(NBUF, NCHUNK)):
            wait_store(NCHUNK - 1 - j)

    return k(x, table, pe2)


def kernel(x, table, pe):
    x = x.astype(jnp.int32)
    pe2 = pe.reshape(SEQ, D)
    return _sc_embed(x, table, pe2)
